# Initial kernel scaffold; baseline (speedup 1.0000x reference)
import jax
import jax.numpy as jnp
from jax.experimental import pallas as pl


def kernel(latent_eval, train_latents):
    raise NotImplementedError("write your pallas kernel here")


# speedup vs baseline: 140.2952x; 140.2952x over previous
import jax
import jax.numpy as jnp
from jax.experimental import pallas as pl
from jax.experimental.pallas import tpu as pltpu

_Q, _K, _D = 1024, 100000, 128
_TEMP = 0.1
_QB = 256
_KB = 1000


def _body(le_ref, tl_ref, out_ref, m_ref, se_ref, swd_ref):
    kj = pl.program_id(0)
    qi = pl.program_id(1)
    nk = pl.num_programs(0)
    row = qi * _QB

    @pl.when(kj == 0)
    def _init():
        m_ref[pl.ds(row, _QB), :] = jnp.full((_QB, 1), jnp.inf, jnp.float32)
        se_ref[pl.ds(row, _QB), :] = jnp.zeros((_QB, 1), jnp.float32)
        swd_ref[pl.ds(row, _QB), :] = jnp.zeros((_QB, 1), jnp.float32)

    le = le_ref[...]
    lq = jnp.sum(le * le, axis=1, keepdims=True)
    le_n = le * jax.lax.rsqrt(jnp.maximum(lq, 1e-24))
    q2 = jnp.sum(le_n * le_n, axis=1, keepdims=True)

    tl = tl_ref[...]
    tq = jnp.sum(tl * tl, axis=1, keepdims=True)
    tl_n = tl * jax.lax.rsqrt(jnp.maximum(tq, 1e-24))

    dot = jax.lax.dot_general(
        le_n.astype(jnp.bfloat16), tl_n.astype(jnp.bfloat16),
        (((1,), (1,)), ((), ())),
        preferred_element_type=jnp.float32)
    d2 = q2 + (1.0 - 2.0 * dot)
    d = jnp.sqrt(jnp.maximum(d2, 1e-12))

    m_old = m_ref[pl.ds(row, _QB), :]
    m_new = jnp.minimum(m_old, jnp.min(d, axis=1, keepdims=True))
    corr = jnp.exp((m_new - m_old) * (1.0 / _TEMP))
    p = jnp.exp((m_new - d) * (1.0 / _TEMP))
    m_ref[pl.ds(row, _QB), :] = m_new
    se_ref[pl.ds(row, _QB), :] = (
        se_ref[pl.ds(row, _QB), :] * corr + jnp.sum(p, axis=1, keepdims=True))
    swd_ref[pl.ds(row, _QB), :] = (
        swd_ref[pl.ds(row, _QB), :] * corr
        + jnp.sum(p * d, axis=1, keepdims=True))

    @pl.when(kj == nk - 1)
    def _fin():
        out_ref[...] = swd_ref[pl.ds(row, _QB), :] / se_ref[pl.ds(row, _QB), :]


def kernel(latent_eval, train_latents):
    out = pl.pallas_call(
        _body,
        grid=(_K // _KB, _Q // _QB),
        in_specs=[
            pl.BlockSpec((_QB, _D), lambda kj, qi: (qi, 0)),
            pl.BlockSpec((_KB, _D), lambda kj, qi: (kj, 0)),
        ],
        out_specs=pl.BlockSpec((_QB, 1), lambda kj, qi: (qi, 0)),
        out_shape=jax.ShapeDtypeStruct((_Q, 1), jnp.float32),
        scratch_shapes=[
            pltpu.VMEM((_Q, 1), jnp.float32),
            pltpu.VMEM((_Q, 1), jnp.float32),
            pltpu.VMEM((_Q, 1), jnp.float32),
        ],
        compiler_params=pltpu.CompilerParams(
            dimension_semantics=("arbitrary", "arbitrary"),
        ),
    )(latent_eval, train_latents)
    return out.reshape(_Q)


# speedup vs baseline: 214.3170x; 1.5276x over previous
import jax
import jax.numpy as jnp
from jax.experimental import pallas as pl
from jax.experimental.pallas import tpu as pltpu

_Q, _K, _D = 1024, 100000, 128
_TEMP = 0.1
_QB = 1024
_KB = 2000


def _body(le_ref, tl_ref, out_ref, m_ref, se_ref, swd_ref):
    kj = pl.program_id(0)
    qi = pl.program_id(1)
    nk = pl.num_programs(0)
    row = qi * _QB

    @pl.when(kj == 0)
    def _init():
        m_ref[pl.ds(row, _QB), :] = jnp.full((_QB, 1), jnp.inf, jnp.float32)
        se_ref[pl.ds(row, _QB), :] = jnp.zeros((_QB, 1), jnp.float32)
        swd_ref[pl.ds(row, _QB), :] = jnp.zeros((_QB, 1), jnp.float32)

    le = le_ref[...]
    lq = jnp.sum(le * le, axis=1, keepdims=True)
    le_n = le * jax.lax.rsqrt(jnp.maximum(lq, 1e-24))
    q2 = jnp.sum(le_n * le_n, axis=1, keepdims=True)

    tl = tl_ref[...]
    tq = jnp.sum(tl * tl, axis=1, keepdims=True)
    tl_n = tl * jax.lax.rsqrt(jnp.maximum(tq, 1e-24))

    dot = jax.lax.dot_general(
        le_n.astype(jnp.bfloat16), tl_n.astype(jnp.bfloat16),
        (((1,), (1,)), ((), ())),
        preferred_element_type=jnp.float32)
    d2 = q2 + (1.0 - 2.0 * dot)
    d = jnp.sqrt(jnp.maximum(d2, 1e-12))

    m_old = m_ref[pl.ds(row, _QB), :]
    m_new = jnp.minimum(m_old, jnp.min(d, axis=1, keepdims=True))
    corr = jnp.exp((m_new - m_old) * (1.0 / _TEMP))
    p = jnp.exp((m_new - d) * (1.0 / _TEMP))
    m_ref[pl.ds(row, _QB), :] = m_new
    se_ref[pl.ds(row, _QB), :] = (
        se_ref[pl.ds(row, _QB), :] * corr + jnp.sum(p, axis=1, keepdims=True))
    swd_ref[pl.ds(row, _QB), :] = (
        swd_ref[pl.ds(row, _QB), :] * corr
        + jnp.sum(p * d, axis=1, keepdims=True))

    @pl.when(kj == nk - 1)
    def _fin():
        out_ref[...] = swd_ref[pl.ds(row, _QB), :] / se_ref[pl.ds(row, _QB), :]


def kernel(latent_eval, train_latents):
    out = pl.pallas_call(
        _body,
        grid=(_K // _KB, _Q // _QB),
        in_specs=[
            pl.BlockSpec((_QB, _D), lambda kj, qi: (qi, 0)),
            pl.BlockSpec((_KB, _D), lambda kj, qi: (kj, 0)),
        ],
        out_specs=pl.BlockSpec((_QB, 1), lambda kj, qi: (qi, 0)),
        out_shape=jax.ShapeDtypeStruct((_Q, 1), jnp.float32),
        scratch_shapes=[
            pltpu.VMEM((_Q, 1), jnp.float32),
            pltpu.VMEM((_Q, 1), jnp.float32),
            pltpu.VMEM((_Q, 1), jnp.float32),
        ],
        compiler_params=pltpu.CompilerParams(
            dimension_semantics=("arbitrary", "arbitrary"),
        ),
    )(latent_eval, train_latents)
    return out.reshape(_Q)


# speedup vs baseline: 363.1874x; 1.6946x over previous
import jax
import jax.numpy as jnp
from jax.experimental import pallas as pl
from jax.experimental.pallas import tpu as pltpu

_Q, _K, _D = 1024, 100000, 128
_TEMP = 0.1
_QB = 1024
_KB = 2000
_NLOG2E_T = -1.4426950408889634 / _TEMP


def _body(le_ref, tl_ref, out_ref, se_ref, swd_ref):
    kj = pl.program_id(0)
    qi = pl.program_id(1)
    nk = pl.num_programs(0)
    row = qi * _QB

    @pl.when(kj == 0)
    def _init():
        se_ref[pl.ds(row, _QB), :] = jnp.zeros((_QB, 1), jnp.float32)
        swd_ref[pl.ds(row, _QB), :] = jnp.zeros((_QB, 1), jnp.float32)

    le = le_ref[...]
    lq = jnp.sum(le * le, axis=1, keepdims=True)
    le_n = le * jax.lax.rsqrt(jnp.maximum(lq, 1e-24))
    q2p = jnp.sum(le_n * le_n, axis=1, keepdims=True) + 1.0

    tl = tl_ref[...]
    tq = jnp.sum(tl * tl, axis=1, keepdims=True)
    tl_n = tl * jax.lax.rsqrt(jnp.maximum(tq, 1e-24))

    dot = jax.lax.dot_general(
        le_n.astype(jnp.bfloat16), tl_n.astype(jnp.bfloat16),
        (((1,), (1,)), ((), ())),
        preferred_element_type=jnp.float32)
    d2 = jnp.maximum(q2p - 2.0 * dot, 1e-12)
    d = d2 * jax.lax.rsqrt(d2)
    p = jnp.exp2(d * _NLOG2E_T)
    se_ref[pl.ds(row, _QB), :] += jnp.sum(p, axis=1, keepdims=True)
    swd_ref[pl.ds(row, _QB), :] += jnp.sum(p * d, axis=1, keepdims=True)

    @pl.when(kj == nk - 1)
    def _fin():
        out_ref[...] = swd_ref[pl.ds(row, _QB), :] / se_ref[pl.ds(row, _QB), :]


def kernel(latent_eval, train_latents):
    out = pl.pallas_call(
        _body,
        grid=(_K // _KB, _Q // _QB),
        in_specs=[
            pl.BlockSpec((_QB, _D), lambda kj, qi: (qi, 0)),
            pl.BlockSpec((_KB, _D), lambda kj, qi: (kj, 0)),
        ],
        out_specs=pl.BlockSpec((_QB, 1), lambda kj, qi: (qi, 0)),
        out_shape=jax.ShapeDtypeStruct((_Q, 1), jnp.float32),
        scratch_shapes=[
            pltpu.VMEM((_Q, 1), jnp.float32),
            pltpu.VMEM((_Q, 1), jnp.float32),
        ],
        compiler_params=pltpu.CompilerParams(
            dimension_semantics=("arbitrary", "arbitrary"),
        ),
    )(latent_eval, train_latents)
    return out.reshape(_Q)
